# baseline (speedup 1.0000x reference)
import jax
import jax.numpy as jnp
from jax import lax
from jax.experimental import pallas as pl
from jax.experimental.pallas import tpu as pltpu
from jax.experimental.pallas import tpu_sc as plsc

NUM_EMB = 1_000_000
DIM = 64
BATCH = 16384
HIST = 50
TOTAL = BATCH * HIST
NC, NS = 2, 16
NW = NC * NS
CHUNK = 256
ROWS_PER_W = TOTAL // NW
NCHUNKS = ROWS_PER_W // CHUNK
VCHUNKS = NUM_EMB // 128
DETILE_STEPS = (VCHUNKS + NW - 1) // NW + 1


def _detile_body(wt_hbm, wtail_hbm, tpad_hbm, s0, s1, o0, o1,
                 is0, is1, os0, os1):
    wid = lax.axis_index("s") * NC + lax.axis_index("c")
    sin = (s0, s1)
    sout = (o0, o1)
    isem = (is0, is1)
    osem = (os0, os1)
    iota16 = lax.iota(jnp.int32, 16)
    rowv = [iota16 + 16 * j for j in range(4)]

    def cid(k):
        return lax.min(wid + NW * k, VCHUNKS - 1)

    def fire_in(k, kb):
        pltpu.async_copy(
            wt_hbm.at[pl.ds(0, DIM), pl.ds(cid(k) * 128, 128)],
            sin[kb], isem[kb])

    def transform(sb, ob):
        def tbody(it, c):
            for q in range(4):
                ip = 4 * it + q
                col = jnp.full((16,), ip, jnp.int32)
                for j in range(4):
                    vals = plsc.load_gather(sb, [rowv[j], col])
                    ob[ip, pl.ds(16 * j, 16)] = vals
            return c
        lax.fori_loop(0, 32, tbody, 0)

    steps = DETILE_STEPS + (DETILE_STEPS % 2)
    fire_in(0, 0)

    def pairstep(p, carry):
        for kb in range(2):
            k = 2 * p + kb
            pltpu.make_async_copy(
                wt_hbm.at[pl.ds(0, DIM), pl.ds(0, 128)], sin[kb],
                isem[kb]).wait()
            fire_in(k + 1, 1 - kb)

            @pl.when(k >= 2)
            def _wait_out(kb=kb):
                pltpu.make_async_copy(
                    sout[kb], tpad_hbm.at[pl.ds(0, 128), pl.ds(0, 128)],
                    osem[kb]).wait()

            transform(sin[kb], sout[kb])
            pltpu.async_copy(
                sout[kb], tpad_hbm.at[pl.ds(cid(k) * 128, 128), pl.ds(0, 128)],
                osem[kb])
        return carry

    lax.fori_loop(0, steps // 2, pairstep, 0)
    pltpu.make_async_copy(
        wt_hbm.at[pl.ds(0, DIM), pl.ds(0, 128)], sin[0], isem[0]).wait()
    for kb in range(2):
        pltpu.make_async_copy(
            sout[kb], tpad_hbm.at[pl.ds(0, 128), pl.ds(0, 128)],
            osem[kb]).wait()

    @pl.when(wid == NW - 1)
    def _rem():
        pltpu.sync_copy(wtail_hbm, o0.at[pl.ds(0, DIM)])
        pltpu.sync_copy(o0.at[pl.ds(0, DIM)],
                        tpad_hbm.at[pl.ds(VCHUNKS * 128, DIM), pl.ds(0, 128)])


def _gather_body(idx_hbm, table_hbm, out_hbm, idx0, idx1, rows0, rows1,
                 si0, si1, sg0, sg1, so0, so1):
    wid = lax.axis_index("s") * NC + lax.axis_index("c")
    u0 = wid * ROWS_PER_W
    idx_v = (idx0, idx1)
    rows_v = (rows0, rows1)
    sem_i = (si0, si1)
    sem_g = (sg0, sg1)
    sem_o = (so0, so1)
    last = u0 + ROWS_PER_W - CHUNK

    def start_idx(g, b):
        base = lax.min(u0 + g * CHUNK, last)
        pltpu.async_copy(idx_hbm.at[pl.ds(base, CHUNK)], idx_v[b], sem_i[b])

    def do_chunk(g, b, wait_out):
        base = u0 + g * CHUNK
        pltpu.make_async_copy(idx_hbm.at[pl.ds(u0, CHUNK)], idx_v[b],
                              sem_i[b]).wait()
        if wait_out:
            pltpu.make_async_copy(rows_v[b].at[pl.ds(0, CHUNK), pl.ds(0, DIM)],
                                  out_hbm.at[pl.ds(base, CHUNK)],
                                  sem_o[b]).wait()
        pltpu.async_copy(table_hbm.at[idx_v[b]], rows_v[b], sem_g[b]).wait()
        start_idx(g + 2, b)
        pltpu.async_copy(rows_v[b].at[pl.ds(0, CHUNK), pl.ds(0, DIM)],
                         out_hbm.at[pl.ds(base, CHUNK)], sem_o[b])

    start_idx(0, 0)
    start_idx(1, 1)
    do_chunk(0, 0, False)
    do_chunk(1, 1, False)

    def pair(i, carry):
        g = 2 * i
        do_chunk(g, 0, True)
        do_chunk(g + 1, 1, True)
        return carry

    lax.fori_loop(1, NCHUNKS // 2, pair, 0)

    for b in range(2):
        pltpu.make_async_copy(rows_v[b].at[pl.ds(0, CHUNK), pl.ds(0, DIM)],
                              out_hbm.at[pl.ds(u0, CHUNK)], sem_o[b]).wait()
        pltpu.make_async_copy(idx_hbm.at[pl.ds(u0, CHUNK)], idx_v[b],
                              sem_i[b]).wait()


@jax.jit
def kernel(token_ids, weight):
    idx = token_ids.T.reshape(TOTAL)
    wt = weight.T
    wtail = jnp.pad(weight[VCHUNKS * 128:, :], ((0, 0), (0, DIM)))
    mesh = plsc.VectorSubcoreMesh(
        core_axis_name="c", subcore_axis_name="s", num_cores=NC, num_subcores=NS
    )

    tpad = pl.kernel(
        _detile_body,
        out_type=jax.ShapeDtypeStruct((NUM_EMB, 128), jnp.float32),
        mesh=mesh,
        scratch_types=[
            pltpu.VMEM((DIM, 128), jnp.float32),
            pltpu.VMEM((DIM, 128), jnp.float32),
            pltpu.VMEM((128, 128), jnp.float32),
            pltpu.VMEM((128, 128), jnp.float32),
            pltpu.SemaphoreType.DMA,
            pltpu.SemaphoreType.DMA,
            pltpu.SemaphoreType.DMA,
            pltpu.SemaphoreType.DMA,
        ],
        compiler_params=pltpu.CompilerParams(
            use_tc_tiling_on_sc=True, needs_layout_passes=False),
    )(wt, wtail)

    out = pl.kernel(
        _gather_body,
        out_type=jax.ShapeDtypeStruct((TOTAL, DIM), jnp.float32),
        mesh=mesh,
        scratch_types=[
            pltpu.VMEM((CHUNK,), jnp.int32),
            pltpu.VMEM((CHUNK,), jnp.int32),
            pltpu.VMEM((CHUNK, 128), jnp.float32),
            pltpu.VMEM((CHUNK, 128), jnp.float32),
            pltpu.SemaphoreType.DMA,
            pltpu.SemaphoreType.DMA,
            pltpu.SemaphoreType.DMA,
            pltpu.SemaphoreType.DMA,
            pltpu.SemaphoreType.DMA,
            pltpu.SemaphoreType.DMA,
        ],
        compiler_params=pltpu.CompilerParams(use_tc_tiling_on_sc=False),
    )(idx, tpad)
    return out.reshape(HIST, BATCH, DIM).transpose(1, 0, 2)


# speedup vs baseline: 1.9147x; 1.9147x over previous
import jax
import jax.numpy as jnp
from jax import lax
from jax.experimental import pallas as pl
from jax.experimental.pallas import tpu as pltpu
from jax.experimental.pallas import tpu_sc as plsc

NUM_EMB = 1_000_000
DIM = 64
BATCH = 16384
HIST = 50
TOTAL = BATCH * HIST
RPS = 128
K = 5
CHUNK = K * RPS
NC, NS = 2, 16
NW = NC * NS
IDX_ROWS = TOTAL // RPS
ROWS_PER_W = IDX_ROWS // NW
NCHUNKS = ROWS_PER_W // K


def _emb_body(idx_hbm, table_hbm, out_hbm, idx0, idx1, rows0, rows1,
              si0, si1, sg0, sg1, so0, so1):
    wid = lax.axis_index("s") * NC + lax.axis_index("c")
    row0 = wid * ROWS_PER_W
    idx_v = (idx0, idx1)
    rows_v = (rows0, rows1)
    sem_i = (si0, si1)
    sem_g = (sg0, sg1)
    sem_o = (so0, so1)
    last = row0 + ROWS_PER_W - K

    def start_idx(g, b):
        base = lax.min(row0 + g * K, last)
        pltpu.async_copy(idx_hbm.at[pl.ds(base * RPS, CHUNK)], idx_v[b], sem_i[b])

    def do_chunk(g, b, wait_out):
        base = row0 + g * K
        pltpu.make_async_copy(idx_hbm.at[pl.ds(row0 * RPS, CHUNK)], idx_v[b],
                              sem_i[b]).wait()
        if wait_out:
            pltpu.make_async_copy(rows_v[b], out_hbm.at[pl.ds(base * RPS, CHUNK)],
                                  sem_o[b]).wait()
        pltpu.async_copy(table_hbm.at[idx_v[b]], rows_v[b], sem_g[b]).wait()
        start_idx(g + 2, b)
        pltpu.async_copy(rows_v[b], out_hbm.at[pl.ds(base * RPS, CHUNK)], sem_o[b])

    start_idx(0, 0)
    start_idx(1, 1)
    do_chunk(0, 0, False)
    do_chunk(1, 1, False)

    def pair(i, carry):
        g = 2 * i
        do_chunk(g, 0, True)
        do_chunk(g + 1, 1, True)
        return carry

    lax.fori_loop(1, NCHUNKS // 2, pair, 0)

    for b in range(2):
        pltpu.make_async_copy(rows_v[b], out_hbm.at[pl.ds(row0 * RPS, CHUNK)],
                              sem_o[b]).wait()
        pltpu.make_async_copy(idx_hbm.at[pl.ds(row0 * RPS, CHUNK)], idx_v[b],
                              sem_i[b]).wait()


@jax.jit
def kernel(token_ids, weight):
    idx = token_ids.T.reshape(TOTAL)
    mesh = plsc.VectorSubcoreMesh(
        core_axis_name="c", subcore_axis_name="s", num_cores=NC, num_subcores=NS
    )
    out = pl.kernel(
        _emb_body,
        out_type=jax.ShapeDtypeStruct((TOTAL, DIM), jnp.float32),
        mesh=mesh,
        scratch_types=[
            pltpu.VMEM((CHUNK,), jnp.int32),
            pltpu.VMEM((CHUNK,), jnp.int32),
            pltpu.VMEM((CHUNK, DIM), jnp.float32),
            pltpu.VMEM((CHUNK, DIM), jnp.float32),
            pltpu.SemaphoreType.DMA,
            pltpu.SemaphoreType.DMA,
            pltpu.SemaphoreType.DMA,
            pltpu.SemaphoreType.DMA,
            pltpu.SemaphoreType.DMA,
            pltpu.SemaphoreType.DMA,
        ],
        compiler_params=pltpu.CompilerParams(use_tc_tiling_on_sc=False),
    )(idx, weight)
    return out.reshape(HIST, BATCH, DIM).transpose(1, 0, 2)
